# Initial kernel scaffold; baseline (speedup 1.0000x reference)
import jax
import jax.numpy as jnp
from jax.experimental import pallas as pl


def kernel(output1, ind1, output2, ind2, mask, mask_cro, ctr_cro_ind, target1, target2, hm_ctxy):
    raise NotImplementedError("write your pallas kernel here")


# speedup vs baseline: 5.7964x; 5.7964x over previous
import functools

import jax
import jax.numpy as jnp
from jax import lax
from jax.experimental import pallas as pl
from jax.experimental.pallas import tpu as pltpu
from jax.experimental.pallas import tpu_sc as plsc

B, C, H, W = 64, 8, 128, 128
HW = H * W
M = 128
N = 128
NTILES = 32
BPT = B // NTILES
EPT = BPT * M * C
NVREG = EPT // 16
NCHUNK = EPT // 128

_mesh = plsc.VectorSubcoreMesh(core_axis_name="c", subcore_axis_name="s")


@functools.partial(
    pl.kernel,
    out_type=jax.ShapeDtypeStruct((NTILES, 16), jnp.float32),
    mesh=_mesh,
    compiler_params=pltpu.CompilerParams(needs_layout_passes=False),
    scratch_types=[
        pltpu.VMEM((BPT * M,), jnp.int32),
        pltpu.VMEM((BPT * N,), jnp.int32),
        pltpu.VMEM((BPT * 4 * M,), jnp.int32),
        pltpu.VMEM((BPT * M,), jnp.int32),
        pltpu.VMEM((BPT * N,), jnp.int32),
        pltpu.VMEM((EPT,), jnp.float32),
        pltpu.VMEM((EPT,), jnp.float32),
        pltpu.VMEM((EPT,), jnp.float32),
        pltpu.VMEM((EPT,), jnp.float32),
        pltpu.VMEM((NCHUNK, 128), jnp.int32),
        pltpu.VMEM((NCHUNK, 128), jnp.int32),
        pltpu.VMEM((16,), jnp.float32),
        pltpu.SemaphoreType.DMA,
    ],
)
def _pair_loss_sc(o1_hbm, i1_hbm, o2_hbm, i2_hbm, mk_hbm, mkc_hbm, ctr_hbm,
                  t1_hbm, t2_hbm, out_hbm,
                  ind1_v, ind2_v, ctr_v, mk_v, mkc_v, t1_v, t2_v,
                  pred1_v, pred2_v, idx1_v, idx2_v, part_v, sem):
    wid = lax.axis_index("s") * 2 + lax.axis_index("c")
    b0 = wid * BPT

    pltpu.sync_copy(i1_hbm.at[pl.ds(pl.multiple_of(b0 * M, 8), BPT * M)], ind1_v)
    pltpu.sync_copy(i2_hbm.at[pl.ds(pl.multiple_of(b0 * N, 8), BPT * N)], ind2_v)

    iota = lax.iota(jnp.int32, 16)

    @pl.loop(0, NVREG)
    def _build(i):
        p = i * 16 + iota
        m = p >> 3
        bb = p >> 10
        ch = p & 7
        base = ((b0 + bb) * C + ch) * HW
        fi1 = base + plsc.load_gather(ind1_v, [m])
        fi2 = base + plsc.load_gather(ind2_v, [m])
        idx1_v[i >> 3, pl.ds((i & 7) * 16, 16)] = fi1
        idx2_v[i >> 3, pl.ds((i & 7) * 16, 16)] = fi2

    copies = []
    for j in range(NCHUNK):
        copies.append(pltpu.async_copy(
            o1_hbm.at[idx1_v.at[j]], pred1_v.at[pl.ds(j * 128, 128)], sem))
        copies.append(pltpu.async_copy(
            o2_hbm.at[idx2_v.at[j]], pred2_v.at[pl.ds(j * 128, 128)], sem))

    pltpu.sync_copy(ctr_hbm.at[pl.ds(pl.multiple_of(b0 * 4 * M, 8), BPT * 4 * M)], ctr_v)
    pltpu.sync_copy(mk_hbm.at[pl.ds(pl.multiple_of(b0 * M, 8), BPT * M)], mk_v)
    pltpu.sync_copy(mkc_hbm.at[pl.ds(pl.multiple_of(b0 * N, 8), BPT * N)], mkc_v)
    pltpu.sync_copy(t1_hbm.at[pl.ds(pl.multiple_of(b0 * M * C, 8), EPT)], t1_v)
    pltpu.sync_copy(t2_hbm.at[pl.ds(pl.multiple_of(b0 * N * C, 8), EPT)], t2_v)

    for cp in copies:
        cp.wait()

    zero = jnp.zeros((16,), jnp.float32)

    @pl.loop(0, NVREG, init_carry=(zero, zero, zero, zero))
    def _compute(i, carry):
        a1, a2, a3, ad = carry
        p = i * 16 + iota
        pm = p >> 3
        bb = p >> 10
        mkf = plsc.load_gather(mk_v, [pm]).astype(jnp.float32)
        mc = plsc.load_gather(mkc_v, [pm])
        cv = plsc.load_gather(ctr_v, [p >> 1])
        src = bb * (4 * N * 2) + 2 * cv + (p & 1)
        p2g = plsc.load_gather(pred2_v, [src])
        t2g = plsc.load_gather(t2_v, [src])
        sl = pl.ds(i * 16, 16)
        p1 = pred1_v[sl]
        t1 = t1_v[sl]
        p2 = pred2_v[sl]
        t2 = t2_v[sl]
        delta = (jnp.abs(p1 - t1) + jnp.abs(p2g - t2g)) / (jnp.abs(t1) + 0.0001)
        delta = delta * delta
        dm = jnp.where(delta > 1.0, 0.0, 1.0)
        delta = delta * dm + (1.0 - dm)
        wgt = 1.0 - jnp.exp(-3.14 * delta)
        mw = mkf * wgt
        a1 = a1 + jnp.abs(p1 * mw - t1 * mw)
        a2 = a2 + jnp.abs(p2g * mw - t2g * mw)
        big = jnp.where((t2 == 0.0).astype(jnp.int32) == mc, 1.0, 0.0)
        a3 = a3 + jnp.abs(p2 * big - t2 * big)
        ad = ad + mkf
        return a1, a2, a3, ad

    a1, a2, a3, ad = _compute
    s1 = jnp.sum(a1)
    s2 = jnp.sum(a2)
    s3 = jnp.sum(a3)
    sd = jnp.sum(ad)
    part = (jnp.where(iota == 0, s1, 0.0) + jnp.where(iota == 1, s2, 0.0)
            + jnp.where(iota == 2, s3, 0.0) + jnp.where(iota == 3, sd, 0.0))
    part_v[...] = part
    pltpu.sync_copy(part_v, out_hbm.at[wid])


def kernel(output1, ind1, output2, ind2, mask, mask_cro, ctr_cro_ind,
           target1, target2, hm_ctxy):
    del hm_ctxy
    parts = _pair_loss_sc(
        output1.reshape(B * C * HW),
        ind1.reshape(B * M),
        output2.reshape(B * C * HW),
        ind2.reshape(B * N),
        mask.reshape(B * M),
        mask_cro.reshape(B * N),
        ctr_cro_ind.reshape(B * 4 * M),
        target1.reshape(B * M * C),
        target2.reshape(B * N * C),
    )
    s = jnp.sum(parts, axis=0)
    denom = s[3] + 0.0001
    loss1 = s[0] / denom
    loss2 = s[1] / denom
    loss3 = s[2] / denom
    return (loss1, 0.5 * loss2 + 0.2 * loss3)
